# baseline (speedup 1.0000x reference)
import functools

import jax
import jax.numpy as jnp
from jax import lax
from jax.experimental import pallas as pl
from jax.experimental.pallas import tpu as pltpu
from jax.experimental.pallas import tpu_sc as plsc

_NEIGHBORS = (
    (-3, 0, 1), (-3, 3, 2), (0, 5, 4), (3, 3, 8),
    (3, 0, 16), (3, -3, 32), (0, -5, 64), (-3, -3, 128),
)

_UNIFORM = (0, 1, 2, 3, 4, 6, 7, 8, 12, 14, 15, 16, 24, 28, 30, 31, 32, 48,
            56, 60, 62, 63, 64, 96, 112, 120, 124, 126, 127, 128, 129, 131,
            135, 143, 159, 191, 192, 193, 195, 199, 207, 223, 224, 225, 227,
            231, 239, 240, 241, 243, 247, 248, 249, 251, 252, 253, 254, 255)

_NUM_CORES = 2
_NUM_SUBCORES = 16
_LANES = 16


_STRIP = 32


def _codes_body(h, w, x_ref, o_ref):
    zero3 = jnp.zeros((3, w), jnp.float32)
    col = lax.broadcasted_iota(jnp.int32, (_STRIP, w), 1)
    cvalids = {dx: (col + dx >= 0) & (col + dx < w)
               for dx in sorted({d[1] for d in _NEIGHBORS if d[1]})}
    packed = jnp.zeros((_STRIP, w), jnp.int32)
    for k in range(h // _STRIP):
        r0 = k * _STRIP
        cur = jnp.floor(x_ref[0, r0:r0 + _STRIP, :] * 255.0)
        prev3 = (zero3 if k == 0
                 else jnp.floor(x_ref[0, r0 - 3:r0, :] * 255.0))
        next3 = (zero3 if r0 + _STRIP == h
                 else jnp.floor(x_ref[0, r0 + _STRIP:r0 + _STRIP + 3, :] * 255.0))
        q_up = jnp.concatenate([prev3, cur[:_STRIP - 3]], axis=0)
        q_dn = jnp.concatenate([cur[3:], next3], axis=0)
        z = jnp.zeros((_STRIP, w), jnp.int32)
        for dy, dx, wt in _NEIGHBORS:
            qd = {-3: q_up, 0: cur, 3: q_dn}[dy]
            if dx:
                rolled = pltpu.roll(qd, (-dx) % w, axis=1)
                nb = jnp.where(cvalids[dx], rolled, 0.0)
            else:
                nb = qd
            z = z + jnp.where(nb >= cur, jnp.int32(wt), jnp.int32(0))
        packed = packed | (z << (8 * (k % 4)))
        if k % 4 == 3:
            o_ref[0, (k // 4) * _STRIP:(k // 4 + 1) * _STRIP, :] = packed
            packed = jnp.zeros((_STRIP, w), jnp.int32)


def _lbp_codes(x):
    b, h, w = x.shape
    return pl.pallas_call(
        functools.partial(_codes_body, h, w),
        grid=(b,),
        in_specs=[pl.BlockSpec((1, h, w), lambda i: (i, 0, 0))],
        out_specs=pl.BlockSpec((1, h // 4, w), lambda i: (i, 0, 0)),
        out_shape=jax.ShapeDtypeStruct((b, h // 4, w), jnp.int32),
    )(x)


_CHUNK = 32768
_UNROLL = 16


def _sc_hist(codes):
    b, hp, w = codes.shape
    rows = _CHUNK // w
    workers = _NUM_CORES * _NUM_SUBCORES
    ipw = b // workers
    chunks_per_img = hp // rows
    nchunks = ipw * chunks_per_img
    mesh = plsc.VectorSubcoreMesh(core_axis_name="c", subcore_axis_name="s")

    @functools.partial(
        pl.kernel, mesh=mesh,
        out_type=jax.ShapeDtypeStruct((b, 256), jnp.int32),
        compiler_params=pltpu.CompilerParams(needs_layout_passes=False),
        scratch_types=[
            pltpu.VMEM((2, rows, w), jnp.int32),
            pltpu.VMEM((_LANES * 256,), jnp.int32),
            pltpu.VMEM((256,), jnp.int32),
            pltpu.SemaphoreType.DMA,
            pltpu.SemaphoreType.DMA,
        ],
    )
    def hist_kernel(codes_hbm, out_hbm, chunk_v, hist_v, red_v, sem0, sem1):
        cid = lax.axis_index("c")
        sid = lax.axis_index("s")
        wid = sid * _NUM_CORES + cid
        img0 = wid * ipw
        laneoff = lax.iota(jnp.int32, _LANES) * 256
        ones = jnp.ones((_LANES,), jnp.int32)
        zeros16 = jnp.zeros((_LANES,), jnp.int32)
        sems = (sem0, sem1)
        per_row = w // _LANES

        def chunk_src(t):
            k, c = divmod(t, chunks_per_img)
            return codes_hbm.at[img0 + k, pl.ds(c * rows, rows), :]

        for j in range(256):
            hist_v[pl.ds(j * _LANES, _LANES)] = zeros16
        copies = [None, None]
        copies[0] = pltpu.async_copy(chunk_src(0), chunk_v.at[0], sems[0])
        for t in range(nchunks):
            buf = t % 2
            if t + 1 < nchunks:
                copies[(t + 1) % 2] = pltpu.async_copy(
                    chunk_src(t + 1), chunk_v.at[(t + 1) % 2], sems[(t + 1) % 2])
            copies[buf].wait()

            @plsc.parallel_loop(0, _CHUNK // _LANES, 1, unroll=_UNROLL)
            def _scatter(i):
                r = i // per_row
                c = i % per_row
                word = chunk_v[buf, r, pl.ds(c * _LANES, _LANES)]
                for sh in (0, 8, 16, 24):
                    code = (word >> sh) & 0xFF
                    plsc.addupdate_scatter(hist_v, [laneoff + code], ones)

            if (t + 1) % chunks_per_img == 0:
                k = t // chunks_per_img
                for j in range(256 // _LANES):
                    acc = hist_v[pl.ds(j * _LANES, _LANES)]
                    for l in range(1, _LANES):
                        acc = acc + hist_v[pl.ds(l * 256 + j * _LANES, _LANES)]
                    red_v[pl.ds(j * _LANES, _LANES)] = acc
                pltpu.sync_copy(red_v, out_hbm.at[img0 + k])
                if t + 1 < nchunks:
                    for j in range(256):
                        hist_v[pl.ds(j * _LANES, _LANES)] = zeros16

    return hist_kernel(codes)


def _remap_body(npix, h_ref, o_ref):
    h = h_ref[...].astype(jnp.float32)
    bsz = h.shape[0]
    ci = lax.broadcasted_iota(jnp.int32, (bsz, 256), 1).astype(jnp.float32)
    present = h > 0.0
    mn = jnp.min(jnp.where(present, ci, 256.0), axis=1, keepdims=True)
    mx = jnp.max(jnp.where(present, ci, -1.0), axis=1, keepdims=True)
    den = mx - mn
    num = 256.0 * (ci - mn)
    qf = jnp.floor(num / den)
    qf = qf - jnp.where(qf * den > num, 1.0, 0.0)
    qf = qf + jnp.where((qf + 1.0) * den <= num, 1.0, 0.0)
    binc = jnp.clip(qf, 0.0, 255.0)
    binc = jnp.where(den > 0.0, binc, 255.0)
    cols = []
    tot = jnp.zeros((bsz,), jnp.float32)
    for u in _UNIFORM:
        uj = jnp.sum(jnp.where(binc == float(u), h, 0.0), axis=1)
        tot = tot + uj
        cols.append(uj)
    cols.append(float(npix) - tot)
    out = jnp.stack(cols, axis=1)
    out = out - 256.0 * jnp.floor(out * (1.0 / 256.0))
    o_ref[...] = out


def _remap(h256, npix):
    b = h256.shape[0]
    return pl.pallas_call(
        functools.partial(_remap_body, npix),
        out_shape=jax.ShapeDtypeStruct((b, 59), jnp.float32),
    )(h256)


def kernel(input):
    b, h, w = input.shape
    codes = _lbp_codes(input)
    h256 = _sc_hist(codes)
    return _remap(h256, h * w)


# speedup vs baseline: 1.1847x; 1.1847x over previous
import functools

import jax
import jax.numpy as jnp
from jax import lax
from jax.experimental import pallas as pl
from jax.experimental.pallas import tpu as pltpu
from jax.experimental.pallas import tpu_sc as plsc

_NEIGHBORS = (
    (-3, 0, 1), (-3, 3, 2), (0, 5, 4), (3, 3, 8),
    (3, 0, 16), (3, -3, 32), (0, -5, 64), (-3, -3, 128),
)

_UNIFORM = (0, 1, 2, 3, 4, 6, 7, 8, 12, 14, 15, 16, 24, 28, 30, 31, 32, 48,
            56, 60, 62, 63, 64, 96, 112, 120, 124, 126, 127, 128, 129, 131,
            135, 143, 159, 191, 192, 193, 195, 199, 207, 223, 224, 225, 227,
            231, 239, 240, 241, 243, 247, 248, 249, 251, 252, 253, 254, 255)

_NUM_CORES = 2
_NUM_SUBCORES = 16
_LANES = 16


_STRIP = 32


def _codes_body(h, w, x_ref, o_ref):
    bf = jnp.bfloat16
    zero3 = jnp.zeros((3, w), bf)
    col = lax.broadcasted_iota(jnp.int32, (_STRIP, w), 1)
    gates = {dx: ((col + dx >= 0) & (col + dx < w)).astype(bf)
             for dx in sorted({d[1] for d in _NEIGHBORS if d[1]})}
    packed = jnp.zeros((_STRIP, w), jnp.int32)
    for k in range(h // _STRIP):
        r0 = k * _STRIP
        cur = jnp.floor(x_ref[0, r0:r0 + _STRIP, :] * 255.0).astype(bf)
        prev3 = (zero3 if k == 0
                 else jnp.floor(x_ref[0, r0 - 3:r0, :] * 255.0).astype(bf))
        next3 = (zero3 if r0 + _STRIP == h
                 else jnp.floor(
                     x_ref[0, r0 + _STRIP:r0 + _STRIP + 3, :] * 255.0).astype(bf))
        q_up = jnp.concatenate([prev3, cur[:_STRIP - 3]], axis=0)
        q_dn = jnp.concatenate([cur[3:], next3], axis=0)
        z = jnp.zeros((_STRIP, w), bf)
        for dy, dx, wt in _NEIGHBORS:
            qd = {-3: q_up, 0: cur, 3: q_dn}[dy]
            if dx:
                nb = pltpu.roll(qd, (-dx) % w, axis=1) * gates[dx]
            else:
                nb = qd
            z = z + jnp.where(nb >= cur, bf(wt), bf(0))
        packed = packed | (z.astype(jnp.int32) << (8 * (k % 4)))
        if k % 4 == 3:
            o_ref[0, (k // 4) * _STRIP:(k // 4 + 1) * _STRIP, :] = packed
            packed = jnp.zeros((_STRIP, w), jnp.int32)


def _lbp_codes(x):
    b, h, w = x.shape
    return pl.pallas_call(
        functools.partial(_codes_body, h, w),
        grid=(b,),
        in_specs=[pl.BlockSpec((1, h, w), lambda i: (i, 0, 0))],
        out_specs=pl.BlockSpec((1, h // 4, w), lambda i: (i, 0, 0)),
        out_shape=jax.ShapeDtypeStruct((b, h // 4, w), jnp.int32),
    )(x)


_CHUNK = 16384
_UNROLL = 8


def _sc_hist(codes):
    b, hp, w = codes.shape
    rows = _CHUNK // w
    workers = _NUM_CORES * _NUM_SUBCORES
    ipw = b // workers
    chunks_per_img = hp // rows
    nchunks = ipw * chunks_per_img
    mesh = plsc.VectorSubcoreMesh(core_axis_name="c", subcore_axis_name="s")

    @functools.partial(
        pl.kernel, mesh=mesh,
        out_type=jax.ShapeDtypeStruct((b, 256), jnp.int32),
        compiler_params=pltpu.CompilerParams(needs_layout_passes=False),
        scratch_types=[
            pltpu.VMEM((2, rows, w), jnp.int32),
            pltpu.VMEM((_LANES * 256,), jnp.int32),
            pltpu.VMEM((256,), jnp.int32),
            pltpu.SemaphoreType.DMA,
            pltpu.SemaphoreType.DMA,
        ],
    )
    def hist_kernel(codes_hbm, out_hbm, chunk_v, hist_v, red_v, sem0, sem1):
        cid = lax.axis_index("c")
        sid = lax.axis_index("s")
        wid = sid * _NUM_CORES + cid
        img0 = wid * ipw
        laneoff = lax.iota(jnp.int32, _LANES) * 256
        ones = jnp.ones((_LANES,), jnp.int32)
        zeros16 = jnp.zeros((_LANES,), jnp.int32)
        sems = (sem0, sem1)
        per_row = w // _LANES

        def chunk_src(t):
            k, c = divmod(t, chunks_per_img)
            return codes_hbm.at[img0 + k, pl.ds(c * rows, rows), :]

        for j in range(256):
            hist_v[pl.ds(j * _LANES, _LANES)] = zeros16
        copies = [None, None]
        copies[0] = pltpu.async_copy(chunk_src(0), chunk_v.at[0], sems[0])
        for t in range(nchunks):
            buf = t % 2
            if t + 1 < nchunks:
                copies[(t + 1) % 2] = pltpu.async_copy(
                    chunk_src(t + 1), chunk_v.at[(t + 1) % 2], sems[(t + 1) % 2])
            copies[buf].wait()

            @plsc.parallel_loop(0, _CHUNK // _LANES, 1, unroll=_UNROLL)
            def _scatter(i):
                r = i // per_row
                c = i % per_row
                word = chunk_v[buf, r, pl.ds(c * _LANES, _LANES)]
                for sh in (0, 8, 16, 24):
                    code = (word >> sh) & 0xFF
                    plsc.addupdate_scatter(hist_v, [laneoff + code], ones)

            if (t + 1) % chunks_per_img == 0:
                k = t // chunks_per_img
                for j in range(256 // _LANES):
                    acc = hist_v[pl.ds(j * _LANES, _LANES)]
                    for l in range(1, _LANES):
                        acc = acc + hist_v[pl.ds(l * 256 + j * _LANES, _LANES)]
                    red_v[pl.ds(j * _LANES, _LANES)] = acc
                pltpu.sync_copy(red_v, out_hbm.at[img0 + k])
                if t + 1 < nchunks:
                    for j in range(256):
                        hist_v[pl.ds(j * _LANES, _LANES)] = zeros16

    return hist_kernel(codes)


def _remap_body(npix, h_ref, o_ref):
    h = h_ref[...].astype(jnp.float32)
    bsz = h.shape[0]
    ci = lax.broadcasted_iota(jnp.int32, (bsz, 256), 1).astype(jnp.float32)
    present = h > 0.0
    mn = jnp.min(jnp.where(present, ci, 256.0), axis=1, keepdims=True)
    mx = jnp.max(jnp.where(present, ci, -1.0), axis=1, keepdims=True)
    den = mx - mn
    num = 256.0 * (ci - mn)
    qf = jnp.floor(num / den)
    qf = qf - jnp.where(qf * den > num, 1.0, 0.0)
    qf = qf + jnp.where((qf + 1.0) * den <= num, 1.0, 0.0)
    binc = jnp.clip(qf, 0.0, 255.0)
    binc = jnp.where(den > 0.0, binc, 255.0)
    cols = []
    tot = jnp.zeros((bsz,), jnp.float32)
    for u in _UNIFORM:
        uj = jnp.sum(jnp.where(binc == float(u), h, 0.0), axis=1)
        tot = tot + uj
        cols.append(uj)
    cols.append(float(npix) - tot)
    out = jnp.stack(cols, axis=1)
    out = out - 256.0 * jnp.floor(out * (1.0 / 256.0))
    o_ref[...] = out


def _remap(h256, npix):
    b = h256.shape[0]
    return pl.pallas_call(
        functools.partial(_remap_body, npix),
        out_shape=jax.ShapeDtypeStruct((b, 59), jnp.float32),
    )(h256)


def kernel(input):
    b, h, w = input.shape
    codes = _lbp_codes(input)
    h256 = _sc_hist(codes)
    return _remap(h256, h * w)
